# baseline (speedup 1.0000x reference)
import functools

import jax
import jax.numpy as jnp
from jax import lax
from jax.experimental import pallas as pl
from jax.experimental.pallas import tpu as pltpu
from jax.experimental.pallas import tpu_sc as plsc

NC = 2
NS = 16
NW = NC * NS

VOCAB = 1000000
DIM = 32
GRP = 4

SLEN = 1024
NBUF = 3


def _transpose_kernel():
    mesh = plsc.VectorSubcoreMesh(core_axis_name="c", subcore_axis_name="s")

    @functools.partial(
        pl.kernel,
        mesh=mesh,
        out_type=jax.ShapeDtypeStruct((VOCAB // 4, 128), jnp.float32),
        scratch_types=[
            pltpu.VMEM((2, GRP, 32, 128), jnp.float32),
            pltpu.VMEM((2, 128, 128), jnp.float32),
            pltpu.SemaphoreType.DMA((2,)),
            pltpu.SemaphoreType.DMA((2,)),
        ],
        compiler_params=pltpu.CompilerParams(
            use_tc_tiling_on_sc=False, needs_layout_passes=False),
    )
    def trans(tab_hbm, out_hbm, vbuf, obuf, isems, osems):
        wid = lax.axis_index("s") * NC + lax.axis_index("c")
        gstart = jnp.where(wid == 0, 0, 62 + (wid - 1) * 61)
        ng = jnp.where(wid == 0, 62, 61)

        lane = lax.broadcasted_iota(jnp.int32, (16,), 0)
        hi = lane + 16

        def in_copies(i, s):
            g = gstart + i
            cps = []
            for c in range(GRP):
                col = (g * GRP + c) * 128
                cps.append(pltpu.make_async_copy(
                    tab_hbm.at[:, pl.ds(col, 128)],
                    vbuf.at[s, c],
                    isems.at[s],
                ))
            return cps

        def out_copy(i, s):
            g = gstart + i
            return pltpu.make_async_copy(
                obuf.at[s], out_hbm.at[pl.ds(g * 128, 128)], osems.at[s])

        def fire(cps):
            for cp in cps:
                cp.start()

        def transpose_group(s):
            for c in range(GRP):
                ref = vbuf.at[s, c]
                for v in range(128):
                    vb = jnp.full((16,), v, jnp.int32)
                    g0 = plsc.load_gather(ref, [lane, vb])
                    g1 = plsc.load_gather(ref, [hi, vb])
                    row = 32 * c + v // 4
                    colo = 32 * (v % 4)
                    obuf[s, row, pl.ds(colo, 16)] = g0
                    obuf[s, row, pl.ds(colo + 16, 16)] = g1

        def step(i, s):
            for cp in in_copies(i, s):
                cp.wait()

            @pl.when(i >= 2)
            def _():
                out_copy(i - 2, s).wait()

            transpose_group(s)
            out_copy(i, s).start()

            @pl.when(i + 2 < ng)
            def _():
                fire(in_copies(i + 2, s))

        fire(in_copies(0, 0))
        fire(in_copies(1, 1))

        def body(j):
            i0 = 2 * j
            step(i0, 0)

            @pl.when(i0 + 1 < ng)
            def _():
                step(i0 + 1, 1)

        pl.loop(0, (ng + 1) // 2)(body)

        @pl.when(wid == 0)
        def _():
            out_copy(ng - 2, 0).wait()
            out_copy(ng - 1, 1).wait()

        @pl.when(wid > 0)
        def _():
            out_copy(ng - 2, 1).wait()
            out_copy(ng - 1, 0).wait()

        @pl.when(wid == NW - 1)
        def _():
            pltpu.make_async_copy(
                tab_hbm.at[:, pl.ds(999936, 64)],
                vbuf.at[0, 0, :, pl.ds(0, 64)],
                isems.at[0],
            ).start()
            pltpu.make_async_copy(
                tab_hbm.at[:, pl.ds(999936, 64)],
                vbuf.at[0, 0, :, pl.ds(0, 64)],
                isems.at[0],
            ).wait()
            ref = vbuf.at[0, 0]
            for v in range(64):
                vb = jnp.full((16,), v, jnp.int32)
                g0 = plsc.load_gather(ref, [lane, vb])
                g1 = plsc.load_gather(ref, [hi, vb])
                row = v // 4
                colo = 32 * (v % 4)
                obuf[0, row, pl.ds(colo, 16)] = g0
                obuf[0, row, pl.ds(colo + 16, 16)] = g1
            pltpu.make_async_copy(
                obuf.at[0, pl.ds(0, 16)],
                out_hbm.at[pl.ds(249984, 16)],
                osems.at[0],
            ).start()
            pltpu.make_async_copy(
                obuf.at[0, pl.ds(0, 16)],
                out_hbm.at[pl.ds(249984, 16)],
                osems.at[0],
            ).wait()

    return trans


def _make_gather(n_rows, dim, slots):
    mesh = plsc.VectorSubcoreMesh(core_axis_name="c", subcore_axis_name="s")

    @functools.partial(
        pl.kernel,
        mesh=mesh,
        out_type=jax.ShapeDtypeStruct((NW, slots, SLEN, dim), jnp.float32),
        scratch_types=[
            pltpu.VMEM((slots, SLEN), jnp.int32),
            pltpu.VMEM((NBUF, SLEN, dim), jnp.float32),
            [pltpu.SemaphoreType.DMA] * NBUF,
            [pltpu.SemaphoreType.DMA] * NBUF,
        ],
        compiler_params=pltpu.CompilerParams(use_tc_tiling_on_sc=False),
    )
    def grab(table_hbm, ids_hbm, out_hbm, idx_v, rows_v, gsems, osems):
        wid = lax.axis_index("s") * NC + lax.axis_index("c")
        pltpu.sync_copy(ids_hbm.at[wid], idx_v)

        def fire_gather(g):
            return pltpu.async_copy(
                table_hbm.at[idx_v.at[g]], rows_v.at[g % NBUF], gsems[g % NBUF]
            )

        gh = {g: fire_gather(g) for g in range(min(2, slots))}
        oh = {}
        for g in range(slots):
            gh.pop(g).wait()
            oh[g] = pltpu.async_copy(
                rows_v.at[g % NBUF], out_hbm.at[wid, g], osems[g % NBUF]
            )
            if g + 2 < slots:
                if g - 1 >= 0:
                    oh.pop(g - 1).wait()
                gh[g + 2] = fire_gather(g + 2)
        for h in oh.values():
            h.wait()

    return grab


def kernel(input, table):
    ids = input
    n = ids.shape[0] * ids.shape[1]
    dim = table.shape[1]
    slots = n // (NW * SLEN)
    ids3 = ids.reshape(NW, slots, SLEN)
    packed = _transpose_kernel()(table.T)
    tlin = packed.reshape(VOCAB, dim)
    out = _make_gather(VOCAB, dim, slots)(tlin, ids3)
    return out.reshape(ids.shape + (dim,))


# speedup vs baseline: 2.4084x; 2.4084x over previous
import functools

import jax
import jax.numpy as jnp
from jax import lax
from jax.experimental import pallas as pl
from jax.experimental.pallas import tpu as pltpu
from jax.experimental.pallas import tpu_sc as plsc

NC = 2
NS = 16
NW = NC * NS

VOCAB = 1000000
DIM = 32
GRP = 4

SLEN = 1024
NBUF = 3


def _tc_transpose():
    BLK = 512

    def body(x_ref, o_ref):
        x = x_ref[...]
        y = jnp.swapaxes(x, 0, 1)
        y4 = y.reshape(BLK // 4, 4, 32)
        o_ref[...] = jnp.concatenate(
            [y4[:, k, :] for k in range(4)], axis=1)

    grid = (1000000 + BLK - 1) // BLK
    return pl.pallas_call(
        body,
        grid=(grid,),
        in_specs=[pl.BlockSpec((32, BLK), lambda g: (0, g))],
        out_specs=pl.BlockSpec((BLK // 4, 128), lambda g: (g, 0)),
        out_shape=jax.ShapeDtypeStruct((250000, 128), jnp.float32),
        compiler_params=pltpu.CompilerParams(
            dimension_semantics=("arbitrary",)),
    )


def _make_gather(n_rows, dim, slots):
    mesh = plsc.VectorSubcoreMesh(core_axis_name="c", subcore_axis_name="s")

    @functools.partial(
        pl.kernel,
        mesh=mesh,
        out_type=jax.ShapeDtypeStruct((NW, slots, SLEN, dim), jnp.float32),
        scratch_types=[
            pltpu.VMEM((slots, SLEN), jnp.int32),
            pltpu.VMEM((NBUF, SLEN, dim), jnp.float32),
            [pltpu.SemaphoreType.DMA] * NBUF,
            [pltpu.SemaphoreType.DMA] * NBUF,
        ],
        compiler_params=pltpu.CompilerParams(use_tc_tiling_on_sc=False),
    )
    def grab(table_hbm, ids_hbm, out_hbm, idx_v, rows_v, gsems, osems):
        wid = lax.axis_index("s") * NC + lax.axis_index("c")
        pltpu.sync_copy(ids_hbm.at[wid], idx_v)

        def fire_gather(g):
            return pltpu.async_copy(
                table_hbm.at[idx_v.at[g]], rows_v.at[g % NBUF], gsems[g % NBUF]
            )

        gh = {g: fire_gather(g) for g in range(min(2, slots))}
        oh = {}
        for g in range(slots):
            gh.pop(g).wait()
            oh[g] = pltpu.async_copy(
                rows_v.at[g % NBUF], out_hbm.at[wid, g], osems[g % NBUF]
            )
            if g + 2 < slots:
                if g - 1 >= 0:
                    oh.pop(g - 1).wait()
                gh[g + 2] = fire_gather(g + 2)
        for h in oh.values():
            h.wait()

    return grab


def kernel(input, table):
    ids = input
    n = ids.shape[0] * ids.shape[1]
    dim = table.shape[1]
    slots = n // (NW * SLEN)
    ids3 = ids.reshape(NW, slots, SLEN)
    packed = _tc_transpose()(table.T)
    tlin = packed.reshape(VOCAB, dim)
    out = _make_gather(VOCAB, dim, slots)(tlin, ids3)
    return out.reshape(ids.shape + (dim,))


# speedup vs baseline: 8.1734x; 3.3937x over previous
import functools

import jax
import jax.numpy as jnp
from jax import lax
from jax.experimental import pallas as pl
from jax.experimental.pallas import tpu as pltpu
from jax.experimental.pallas import tpu_sc as plsc

NC = 2
NS = 16
NW = NC * NS

VOCAB = 1000000
DIM = 32
GRP = 4

SLEN = 1024
NBUF = 3


def _tc_transpose():
    B4 = 2048
    NB = 123

    def body(x0, x1, x2, x3, o_ref):
        x = jnp.concatenate(
            [x0[...], x1[...], x2[...], x3[...]], axis=0)
        eye = jnp.eye(128, dtype=jnp.float32)
        o_ref[...] = jax.lax.dot_general(
            x, eye, (((0,), (0,)), ((), ())),
            preferred_element_type=jnp.float32)

    def in_spec(m):
        return pl.BlockSpec(
            (32, B4), lambda g, m=m: (0, jnp.minimum(g + m * NB, 488)))

    return pl.pallas_call(
        body,
        grid=(NB,),
        in_specs=[in_spec(m) for m in range(4)],
        out_specs=pl.BlockSpec((B4, 128), lambda g: (g, 0)),
        out_shape=jax.ShapeDtypeStruct((NB * B4, 128), jnp.float32),
        compiler_params=pltpu.CompilerParams(
            dimension_semantics=("arbitrary",)),
    )


def _make_gather(n_rows, dim, slots):
    mesh = plsc.VectorSubcoreMesh(core_axis_name="c", subcore_axis_name="s")

    @functools.partial(
        pl.kernel,
        mesh=mesh,
        out_type=jax.ShapeDtypeStruct((NW, slots, SLEN, dim), jnp.float32),
        scratch_types=[
            pltpu.VMEM((slots, SLEN), jnp.int32),
            pltpu.VMEM((NBUF, SLEN, dim), jnp.float32),
            [pltpu.SemaphoreType.DMA] * NBUF,
            [pltpu.SemaphoreType.DMA] * NBUF,
        ],
        compiler_params=pltpu.CompilerParams(use_tc_tiling_on_sc=False),
    )
    def grab(table_hbm, ids_hbm, out_hbm, idx_v, rows_v, gsems, osems):
        wid = lax.axis_index("s") * NC + lax.axis_index("c")
        pltpu.sync_copy(ids_hbm.at[wid], idx_v)

        def fire_gather(g):
            return pltpu.async_copy(
                table_hbm.at[idx_v.at[g]], rows_v.at[g % NBUF], gsems[g % NBUF]
            )

        gh = {g: fire_gather(g) for g in range(min(2, slots))}
        oh = {}
        for g in range(slots):
            gh.pop(g).wait()
            oh[g] = pltpu.async_copy(
                rows_v.at[g % NBUF], out_hbm.at[wid, g], osems[g % NBUF]
            )
            if g + 2 < slots:
                if g - 1 >= 0:
                    oh.pop(g - 1).wait()
                gh[g + 2] = fire_gather(g + 2)
        for h in oh.values():
            h.wait()

    return grab


def kernel(input, table):
    ids = input
    n = ids.shape[0] * ids.shape[1]
    dim = table.shape[1]
    slots = n // (NW * SLEN)
    flat = ids.reshape(-1)
    flat = 4 * (flat % 251904) + flat // 251904
    ids3 = flat.reshape(NW, slots, SLEN)
    tt = table.T
    packed = _tc_transpose()(tt, tt, tt, tt)
    tlin = packed.reshape(4 * 251904, dim)
    out = _make_gather(4 * 251904, dim, slots)(tlin, ids3)
    return out.reshape(ids.shape + (dim,))
